# baseline (speedup 1.0000x reference)
import functools

import jax
import jax.numpy as jnp
from jax import lax
from jax.experimental import pallas as pl
from jax.experimental.pallas import tpu as pltpu
from jax.experimental.pallas import tpu_sc as plsc

VOCAB = 1000000
D = 64
BATCH = 4096
HIST = 200

NC, NS = 2, 16
NW = NC * NS
CB = 128
HG = 8
N_HG = HIST // HG


def _build():
  mesh = plsc.VectorSubcoreMesh(
      core_axis_name="c", subcore_axis_name="s", num_cores=NC, num_subcores=NS)

  @functools.partial(
      pl.kernel,
      mesh=mesh,
      out_type=jax.ShapeDtypeStruct((HIST, D, BATCH), jnp.float32),
      scratch_types=[
          pltpu.VMEM((HG, CB), jnp.int32),
          pltpu.VMEM((CB, 2 * D), jnp.float32),
          pltpu.VMEM((CB, 2 * D), jnp.float32),
          pltpu.VMEM((D, CB), jnp.float32),
          pltpu.VMEM((D, CB), jnp.float32),
          pltpu.SemaphoreType.DMA,
          pltpu.SemaphoreType.DMA,
          pltpu.SemaphoreType.DMA,
          pltpu.SemaphoreType.DMA,
      ],
      compiler_params=pltpu.CompilerParams(needs_layout_passes=False),
  )
  def emb_kernel(tokT_hbm, tab_hbm, out_hbm, idx_v, bufa, bufb, obufa, obufb,
                 gsa, gsb, osa, osb):
    wid = lax.axis_index("s") * NC + lax.axis_index("c")
    col0 = pl.multiple_of(wid * CB, CB)
    lanes = lax.iota(jnp.int32, 16)

    bufs = (bufa, bufb)
    gsems = (gsa, gsb)
    obufs = (obufa, obufb)
    osems = (osa, osb)

    def transpose_block(buf, obuf):
      def tbody(d, carry):
        dcol = jnp.full((16,), d, jnp.int32)
        for g in range(CB // 16):
          rows = lanes + g * 16
          vals = plsc.load_gather(buf, [rows, dcol])
          plsc.store_scatter(obuf, [dcol, rows], vals)
        return carry
      lax.fori_loop(0, D, tbody, 0)

    def fire_gather(h, p):
      return pltpu.async_copy(tab_hbm.at[idx_v.at[h]], bufs[p], gsems[p])

    def make_unit(first):
      def unit(hg, carry):
        h0 = pl.multiple_of(hg * HG, HG)
        pltpu.sync_copy(tokT_hbm.at[pl.ds(h0, HG), pl.ds(col0, CB)], idx_v)
        fire_gather(0, 0)
        for h in range(HG):
          p = h % 2
          if h + 1 < HG:
            fire_gather(h + 1, 1 - p)
          pltpu.make_async_copy(
              tab_hbm.at[idx_v.at[h]], bufs[p], gsems[p]).wait()
          if not (first and h < 2):
            pltpu.make_async_copy(
                obufs[p], out_hbm.at[0, pl.ds(0, D), pl.ds(col0, CB)],
                osems[p]).wait()
          transpose_block(bufs[p], obufs[p])
          pltpu.async_copy(
              obufs[p], out_hbm.at[h0 + h, pl.ds(0, D), pl.ds(col0, CB)],
              osems[p])
        return carry
      return unit

    make_unit(True)(0, 0)
    lax.fori_loop(1, N_HG, make_unit(False), 0)
    for p in range(2):
      pltpu.make_async_copy(
          obufs[p], out_hbm.at[0, pl.ds(0, D), pl.ds(col0, CB)],
          osems[p]).wait()

  return emb_kernel


_emb = _build()


def kernel(token_ids, table):
  tab128 = jnp.concatenate([table, jnp.zeros_like(table)], axis=1)
  tokT = token_ids.T.astype(jnp.int32)
  outP = _emb(tokT, tab128)
  return outP.transpose(2, 0, 1)


# speedup vs baseline: 1.1442x; 1.1442x over previous
import functools

import jax
import jax.numpy as jnp
from jax import lax
from jax.experimental import pallas as pl
from jax.experimental.pallas import tpu as pltpu
from jax.experimental.pallas import tpu_sc as plsc

VOCAB = 1000000
D = 64
BATCH = 4096
HIST = 200

NC, NS = 2, 16
NW = NC * NS
CB = 128
HG = 8
N_HG = HIST // HG


def _build():
  mesh = plsc.VectorSubcoreMesh(
      core_axis_name="c", subcore_axis_name="s", num_cores=NC, num_subcores=NS)

  @functools.partial(
      pl.kernel,
      mesh=mesh,
      out_type=jax.ShapeDtypeStruct((HIST, D, BATCH), jnp.float32),
      scratch_types=[
          pltpu.VMEM((HG, CB), jnp.int32),
          pltpu.VMEM((CB, 2 * D), jnp.float32),
          pltpu.VMEM((CB, 2 * D), jnp.float32),
          pltpu.VMEM((D, CB), jnp.float32),
          pltpu.VMEM((D, CB), jnp.float32),
          pltpu.SemaphoreType.DMA,
          pltpu.SemaphoreType.DMA,
          pltpu.SemaphoreType.DMA,
          pltpu.SemaphoreType.DMA,
      ],
      compiler_params=pltpu.CompilerParams(needs_layout_passes=False),
  )
  def emb_kernel(tokT_hbm, tab_hbm, out_hbm, idx_v, bufa, bufb, obufa, obufb,
                 gsa, gsb, osa, osb):
    wid = lax.axis_index("s") * NC + lax.axis_index("c")
    col0 = pl.multiple_of(wid * CB, CB)
    lanes = lax.iota(jnp.int32, 16)

    bufs = (bufa, bufb)
    gsems = (gsa, gsb)
    obufs = (obufa, obufb)
    osems = (osa, osb)

    rowvecs = [lanes + 16 * g for g in range(D // 16)]

    def transpose_block(buf, obuf):
      def tbody(t, carry):
        tcol = jnp.full((16,), t, jnp.int32)
        for g in range(D // 16):
          vals = buf[t, pl.ds(16 * g, 16)]
          plsc.store_scatter(obuf, [rowvecs[g], tcol], vals)
        return carry
      lax.fori_loop(0, CB, tbody, 0, unroll=8)

    def fire_gather(h, p):
      return pltpu.async_copy(tab_hbm.at[idx_v.at[h]], bufs[p], gsems[p])

    def make_unit(first):
      def unit(hg, carry):
        h0 = pl.multiple_of(hg * HG, HG)
        pltpu.sync_copy(tokT_hbm.at[pl.ds(h0, HG), pl.ds(col0, CB)], idx_v)
        fire_gather(0, 0)
        for h in range(HG):
          p = h % 2
          if h + 1 < HG:
            fire_gather(h + 1, 1 - p)
          pltpu.make_async_copy(
              tab_hbm.at[idx_v.at[h]], bufs[p], gsems[p]).wait()
          if not (first and h < 2):
            pltpu.make_async_copy(
                obufs[p], out_hbm.at[0, pl.ds(0, D), pl.ds(col0, CB)],
                osems[p]).wait()
          transpose_block(bufs[p], obufs[p])
          pltpu.async_copy(
              obufs[p], out_hbm.at[h0 + h, pl.ds(0, D), pl.ds(col0, CB)],
              osems[p])
        return carry
      return unit

    make_unit(True)(0, 0)
    lax.fori_loop(1, N_HG, make_unit(False), 0)
    for p in range(2):
      pltpu.make_async_copy(
          obufs[p], out_hbm.at[0, pl.ds(0, D), pl.ds(col0, CB)],
          osems[p]).wait()

  return emb_kernel


_emb = _build()


def kernel(token_ids, table):
  tab128 = jnp.concatenate([table, jnp.zeros_like(table)], axis=1)
  tokT = token_ids.T.astype(jnp.int32)
  outP = _emb(tokT, tab128)
  return outP.transpose(2, 0, 1)


# speedup vs baseline: 1.3982x; 1.2220x over previous
import functools

import jax
import jax.numpy as jnp
from jax import lax
from jax.experimental import pallas as pl
from jax.experimental.pallas import tpu as pltpu
from jax.experimental.pallas import tpu_sc as plsc

VOCAB = 1000000
D = 64
BATCH = 4096
HIST = 200

NC, NS = 2, 16
NW = NC * NS
CB = 128
HG = 8
N_HG = HIST // HG


def _build():
  mesh = plsc.VectorSubcoreMesh(
      core_axis_name="c", subcore_axis_name="s", num_cores=NC, num_subcores=NS)

  @functools.partial(
      pl.kernel,
      mesh=mesh,
      out_type=jax.ShapeDtypeStruct((HIST, D, BATCH), jnp.float32),
      scratch_types=[
          pltpu.VMEM((HG, CB), jnp.int32),
          pltpu.VMEM((CB, 2 * D), jnp.float32),
          pltpu.VMEM((CB, 2 * D), jnp.float32),
          pltpu.VMEM((D, CB), jnp.float32),
          pltpu.VMEM((D, CB), jnp.float32),
          pltpu.SemaphoreType.DMA,
          pltpu.SemaphoreType.DMA,
          pltpu.SemaphoreType.DMA,
          pltpu.SemaphoreType.DMA,
      ],
      compiler_params=pltpu.CompilerParams(needs_layout_passes=False),
  )
  def emb_kernel(tokT_hbm, tab_hbm, out_hbm, idx_v, bufa, bufb, obufa, obufb,
                 gsa, gsb, osa, osb):
    wid = lax.axis_index("s") * NC + lax.axis_index("c")
    col0 = pl.multiple_of(wid * CB, CB)
    lanes = lax.iota(jnp.int32, 16)

    bufs = (bufa, bufb)
    gsems = (gsa, gsb)
    obufs = (obufa, obufb)
    osems = (osa, osb)

    rowvecs = [lanes + 16 * g for g in range(D // 16)]

    def transpose_block(buf, obuf):
      @plsc.parallel_loop(0, CB, 1, unroll=8)
      def tbody(t):
        tcol = jnp.full((16,), t, jnp.int32)
        for g in range(D // 16):
          vals = buf[t, pl.ds(16 * g, 16)]
          plsc.store_scatter(obuf, [rowvecs[g], tcol], vals)

    def fire_gather(h, p):
      return pltpu.async_copy(tab_hbm.at[idx_v.at[h]], bufs[p], gsems[p])

    def make_unit(first):
      def unit(hg, carry):
        h0 = pl.multiple_of(hg * HG, HG)
        pltpu.sync_copy(tokT_hbm.at[pl.ds(h0, HG), pl.ds(col0, CB)], idx_v)
        fire_gather(0, 0)
        for h in range(HG):
          p = h % 2
          if h + 1 < HG:
            fire_gather(h + 1, 1 - p)
          pltpu.make_async_copy(
              tab_hbm.at[idx_v.at[h]], bufs[p], gsems[p]).wait()
          if not (first and h < 2):
            pltpu.make_async_copy(
                obufs[p], out_hbm.at[0, pl.ds(0, D), pl.ds(col0, CB)],
                osems[p]).wait()
          transpose_block(bufs[p], obufs[p])
          pltpu.async_copy(
              obufs[p], out_hbm.at[h0 + h, pl.ds(0, D), pl.ds(col0, CB)],
              osems[p])
        return carry
      return unit

    make_unit(True)(0, 0)
    lax.fori_loop(1, N_HG, make_unit(False), 0)
    for p in range(2):
      pltpu.make_async_copy(
          obufs[p], out_hbm.at[0, pl.ds(0, D), pl.ds(col0, CB)],
          osems[p]).wait()

  return emb_kernel


_emb = _build()


def kernel(token_ids, table):
  tab128 = jnp.concatenate([table, jnp.zeros_like(table)], axis=1)
  tokT = token_ids.T.astype(jnp.int32)
  outP = _emb(tokT, tab128)
  return outP.transpose(2, 0, 1)


# speedup vs baseline: 3.0677x; 2.1940x over previous
import functools

import jax
import jax.numpy as jnp
from jax import lax
from jax.experimental import pallas as pl
from jax.experimental.pallas import tpu as pltpu
from jax.experimental.pallas import tpu_sc as plsc

VOCAB = 1000000
D = 64
BATCH = 4096
HIST = 200

NC, NS = 2, 16
NW = NC * NS
CB = 128
HG = 8
N_HG = HIST // HG

_MESH = plsc.VectorSubcoreMesh(
    core_axis_name="c", subcore_axis_name="s", num_cores=NC, num_subcores=NS)


def _build_packT():
  C = 384
  MAIN = 999936
  NBLK = MAIN // C
  NJ = (NBLK + 2 * NW - 1) // (2 * NW)

  @functools.partial(
      pl.kernel,
      mesh=_MESH,
      out_type=jax.ShapeDtypeStruct((VOCAB // 2, 2 * D), jnp.float32),
      scratch_types=[
          pltpu.VMEM((D, C), jnp.float32),
          pltpu.VMEM((D, C), jnp.float32),
          pltpu.VMEM((C // 2, 2 * D), jnp.float32),
          pltpu.VMEM((C // 2, 2 * D), jnp.float32),
          pltpu.VMEM((D, D), jnp.float32),
          pltpu.SemaphoreType.DMA,
          pltpu.SemaphoreType.DMA,
          pltpu.SemaphoreType.DMA,
          pltpu.SemaphoreType.DMA,
      ],
      compiler_params=pltpu.CompilerParams(needs_layout_passes=False),
  )
  def packT_kernel(tabT_hbm, tail_hbm, out_hbm, ina, inb, oba, obb, tailb,
                   isa, isb, osa, osb):
    wid = lax.axis_index("s") * NC + lax.axis_index("c")
    lanes = lax.iota(jnp.int32, 16)
    inbufs, isems = (ina, inb), (isa, isb)
    obufs, osems = (oba, obb), (osa, osb)
    tvecs = [lanes + 16 * g for g in range(C // 16)]
    trows = [v >> 1 for v in tvecs]
    tcols = [(v & 1) * D for v in tvecs]

    def transpose_pack(inbuf, obuf, ngroups):
      @plsc.parallel_loop(0, D, 1, unroll=4)
      def tb(s):
        dvec = (lanes + s) & (D - 1)
        for g in range(ngroups):
          vals = plsc.load_gather(inbuf, [dvec, tvecs[g]])
          plsc.store_scatter(obuf, [trows[g], dvec + tcols[g]], vals)

    def blk(c, p, first):
      @pl.when(c < NBLK)
      def _():
        coff = pl.multiple_of(c * C, 128)
        loff = pl.multiple_of(c * (C // 2), 8)
        if not first:
          pltpu.make_async_copy(
              obufs[p], out_hbm.at[pl.ds(0, C // 2)], osems[p]).wait()
        pltpu.async_copy(
            tabT_hbm.at[pl.ds(0, D), pl.ds(coff, C)], inbufs[p],
            isems[p]).wait()
        transpose_pack(inbufs[p], obufs[p], C // 16)
        pltpu.async_copy(
            obufs[p], out_hbm.at[pl.ds(loff, C // 2)], osems[p])

    def step(j, carry, first=False):
      c0 = wid + 2 * NW * j
      blk(c0, 0, first)
      blk(c0 + NW, 1, first)
      return carry

    step(0, 0, first=True)
    lax.fori_loop(1, NJ, step, 0)
    for p in range(2):
      pltpu.make_async_copy(
          obufs[p], out_hbm.at[pl.ds(0, C // 2)], osems[p]).wait()

    @pl.when(wid == 0)
    def _():
      pltpu.sync_copy(tail_hbm, tailb)
      transpose_pack(tailb, obufs[0], D // 16)
      pltpu.sync_copy(obufs[0].at[pl.ds(0, 32)],
                      out_hbm.at[pl.ds(MAIN // 2, 32)])

  return packT_kernel


def _build():
  @functools.partial(
      pl.kernel,
      mesh=_MESH,
      out_type=jax.ShapeDtypeStruct((HIST, D, BATCH), jnp.float32),
      scratch_types=[
          pltpu.VMEM((HG, CB), jnp.int32),
          pltpu.VMEM((HG, CB), jnp.int32),
          pltpu.VMEM((CB, 2 * D), jnp.float32),
          pltpu.VMEM((CB, 2 * D), jnp.float32),
          pltpu.VMEM((D, CB), jnp.float32),
          pltpu.VMEM((D, CB), jnp.float32),
          pltpu.SemaphoreType.DMA,
          pltpu.SemaphoreType.DMA,
          pltpu.SemaphoreType.DMA,
          pltpu.SemaphoreType.DMA,
      ],
      compiler_params=pltpu.CompilerParams(needs_layout_passes=False),
  )
  def emb_kernel(tokT_hbm, tab_hbm, out_hbm, idx_v, idx2_v, bufa, bufb,
                 obufa, obufb, gsa, gsb, osa, osb):
    wid = lax.axis_index("s") * NC + lax.axis_index("c")
    col0 = pl.multiple_of(wid * CB, CB)
    lanes = lax.iota(jnp.int32, 16)

    bufs = (bufa, bufb)
    gsems = (gsa, gsb)
    obufs = (obufa, obufb)
    osems = (osa, osb)

    rowvecs = [lanes + 16 * g for g in range(CB // 16)]

    def transpose_block(h, buf, obuf):
      parvecs = [(idx_v[h, pl.ds(16 * g, 16)] & 1) * D
                 for g in range(CB // 16)]

      @plsc.parallel_loop(0, D, 1, unroll=4)
      def tbody(s):
        dvec = (lanes + s) & (D - 1)
        for g in range(CB // 16):
          vals = plsc.load_gather(buf, [rowvecs[g], dvec + parvecs[g]])
          plsc.store_scatter(obuf, [dvec, rowvecs[g]], vals)

    def fire_gather(h, p):
      return pltpu.async_copy(tab_hbm.at[idx2_v.at[h]], bufs[p], gsems[p])

    def make_unit(first):
      def unit(hg, carry):
        h0 = pl.multiple_of(hg * HG, HG)
        pltpu.sync_copy(tokT_hbm.at[pl.ds(h0, HG), pl.ds(col0, CB)], idx_v)
        for h in range(HG):
          for g in range(CB // 16):
            idx2_v[h, pl.ds(16 * g, 16)] = (
                idx_v[h, pl.ds(16 * g, 16)] >> 1)
        fire_gather(0, 0)
        for h in range(HG):
          p = h % 2
          if h + 1 < HG:
            fire_gather(h + 1, 1 - p)
          pltpu.make_async_copy(
              tab_hbm.at[idx2_v.at[h]], bufs[p], gsems[p]).wait()
          if not (first and h < 2):
            pltpu.make_async_copy(
                obufs[p], out_hbm.at[0, pl.ds(0, D), pl.ds(col0, CB)],
                osems[p]).wait()
          transpose_block(h, bufs[p], obufs[p])
          pltpu.async_copy(
              obufs[p], out_hbm.at[h0 + h, pl.ds(0, D), pl.ds(col0, CB)],
              osems[p])
        return carry
      return unit

    make_unit(True)(0, 0)
    lax.fori_loop(1, N_HG, make_unit(False), 0)
    for p in range(2):
      pltpu.make_async_copy(
          obufs[p], out_hbm.at[0, pl.ds(0, D), pl.ds(col0, CB)],
          osems[p]).wait()

  return emb_kernel


_packT = _build_packT()
_emb = _build()


def kernel(token_ids, table):
  tabT = table.T
  tail = tabT[:, VOCAB - D:]
  tab128 = _packT(tabT, tail)
  tokT = token_ids.T.astype(jnp.int32)
  outP = _emb(tokT, tab128)
  return outP.transpose(2, 0, 1)
